# baseline (speedup 1.0000x reference)
import functools

import jax
import jax.numpy as jnp
from jax import lax
from jax.experimental import pallas as pl
from jax.experimental.pallas import tpu as pltpu
from jax.experimental.pallas import tpu_sc as plsc

_B, _C, _N = 16, 85, 20000
_L = 16
_W = 1280
_WLAST = 896

_mesh = plsc.VectorSubcoreMesh(core_axis_name="c", subcore_axis_name="s")


@functools.partial(
    pl.kernel,
    mesh=_mesh,
    out_type=jax.ShapeDtypeStruct((_B, _N), jnp.float32),
    scratch_types=[
        pltpu.VMEM((8, 8, _W), jnp.float32),
        pltpu.VMEM((8, _W), jnp.float32),
    ],
)
def _sc_threshold(x_hbm, out_hbm, ibuf, obuf):
    g = lax.axis_index("c")
    k = lax.axis_index("s")
    row = g * 8
    col = k * _W

    def run(w):
        pltpu.sync_copy(
            x_hbm.at[pl.ds(row, 8), pl.ds(0, 8), pl.ds(col, w)],
            ibuf.at[:, :, pl.ds(0, w)])

        def body(i, carry):
            j = i // (w // _L)
            v = i % (w // _L)
            x = ibuf[j, 4, pl.ds(v * _L, _L)]
            obuf[j, pl.ds(v * _L, _L)] = jnp.where(
                x > jnp.float32(0.15), jnp.float32(0.0), x)
            return carry

        lax.fori_loop(0, 8 * (w // _L), body, 0)
        pltpu.sync_copy(
            obuf.at[:, pl.ds(0, w)],
            out_hbm.at[pl.ds(row, 8), pl.ds(col, w)])

    @pl.when(k < 15)
    def _():
        run(_W)

    @pl.when(k == 15)
    def _():
        run(_WLAST)


@jax.jit
def kernel(output):
    B, C, N = output.shape
    scores = _sc_threshold(output)
    boxes = jnp.zeros((B, N, 4), jnp.int32)
    n = jnp.asarray(B, dtype=jnp.int32)
    return (n, boxes, scores)


# speedup vs baseline: 4.1046x; 4.1046x over previous
import functools

import jax
import jax.numpy as jnp
from jax import lax
from jax.experimental import pallas as pl
from jax.experimental.pallas import tpu as pltpu
from jax.experimental.pallas import tpu_sc as plsc

_NC = 2
_NS = 16
_L = 16
_TOTAL = 16 * 20000
_PER_W = _TOTAL // (_NC * _NS)

_mesh = plsc.VectorSubcoreMesh(core_axis_name="c", subcore_axis_name="s")


@functools.partial(
    pl.kernel,
    mesh=_mesh,
    out_type=jax.ShapeDtypeStruct((_TOTAL,), jnp.float32),
    scratch_types=[pltpu.VMEM((_PER_W,), jnp.float32)],
)
def _sc_threshold(conf_hbm, out_hbm, buf):
    wid = lax.axis_index("s") * _NC + lax.axis_index("c")
    base = wid * _PER_W
    pltpu.sync_copy(conf_hbm.at[pl.ds(base, _PER_W)], buf)

    @plsc.parallel_loop(0, _PER_W // _L, unroll=5)
    def body(i):
        v = buf[pl.ds(i * _L, _L)]
        buf[pl.ds(i * _L, _L)] = jnp.where(
            v > jnp.float32(0.15), jnp.float32(0.0), v)
    pltpu.sync_copy(buf, out_hbm.at[pl.ds(base, _PER_W)])


@jax.jit
def kernel(output):
    B, C, N = output.shape
    conf = output[:, 4, :].reshape(B * N)
    scores = _sc_threshold(conf).reshape(B, N)
    boxes = jnp.zeros((B, N, 4), jnp.int32)
    n = jnp.asarray(B, dtype=jnp.int32)
    return (n, boxes, scores)


# speedup vs baseline: 4.2164x; 1.0272x over previous
import functools

import jax
import jax.numpy as jnp
from jax import lax
from jax.experimental import pallas as pl
from jax.experimental.pallas import tpu as pltpu
from jax.experimental.pallas import tpu_sc as plsc

_B, _N = 16, 20000
_L = 16
_W = 1280
_WLAST = 896

_mesh = plsc.VectorSubcoreMesh(core_axis_name="c", subcore_axis_name="s")


@functools.partial(
    pl.kernel,
    mesh=_mesh,
    out_type=jax.ShapeDtypeStruct((_B, _N), jnp.float32),
    scratch_types=[pltpu.VMEM((8, _W), jnp.float32)],
)
def _sc_threshold(conf_hbm, out_hbm, buf):
    g = lax.axis_index("c")
    k = lax.axis_index("s")
    row = g * 8
    col = k * _W

    def run(w):
        pltpu.sync_copy(
            conf_hbm.at[pl.ds(row, 8), pl.ds(col, w)],
            buf.at[:, pl.ds(0, w)])

        @plsc.parallel_loop(0, 8 * (w // _L), unroll=5)
        def body(i):
            j = i // (w // _L)
            v = i % (w // _L)
            x = buf[j, pl.ds(v * _L, _L)]
            buf[j, pl.ds(v * _L, _L)] = jnp.where(
                x > jnp.float32(0.15), jnp.float32(0.0), x)

        pltpu.sync_copy(
            buf.at[:, pl.ds(0, w)],
            out_hbm.at[pl.ds(row, 8), pl.ds(col, w)])

    @pl.when(k < 15)
    def _():
        run(_W)

    @pl.when(k == 15)
    def _():
        run(_WLAST)


@jax.jit
def kernel(output):
    B, C, N = output.shape
    conf = output[:, 4, :]
    scores = _sc_threshold(conf)
    boxes = jnp.zeros((B, N, 4), jnp.int32)
    n = jnp.asarray(B, dtype=jnp.int32)
    return (n, boxes, scores)
